# speedup vs baseline: 18.7359x; 18.7359x over previous
import jax
import jax.numpy as jnp
from jax.experimental import pallas as pl

_BB = 256


def _embed_block(x_ref, w_ref, o_ref):
    xb = x_ref[...]
    xb = jnp.where(jnp.isnan(xb), 0.0, xb)
    w = w_ref[...]
    o_ref[...] = xb[:, :, None] * w[None, :, :]


def kernel(x, emb_weight):
    B, L = x.shape
    D = emb_weight.shape[1]
    w = emb_weight[1:L + 1]
    grid = (B // _BB,)
    return pl.pallas_call(
        _embed_block,
        grid=grid,
        in_specs=[
            pl.BlockSpec((_BB, L), lambda i: (i, 0)),
            pl.BlockSpec((L, D), lambda i: (0, 0)),
        ],
        out_specs=pl.BlockSpec((_BB, L, D), lambda i: (i, 0, 0)),
        out_shape=jax.ShapeDtypeStruct((B, L, D), x.dtype),
    )(x, w)


# speedup vs baseline: 22.9656x; 1.2258x over previous
import jax
import jax.numpy as jnp
from jax.experimental import pallas as pl

_BB = 256


def _embed_block(xl_ref, xr_ref, w_ref, o_ref):
    bb, h, d2 = o_ref.shape
    d = d2 // 2
    xl = xl_ref[...]
    xr = xr_ref[...]
    xl = jnp.where(jnp.isnan(xl), 0.0, xl)
    xr = jnp.where(jnp.isnan(xr), 0.0, xr)
    xlb = jax.lax.broadcast_in_dim(xl, (bb, h, d2), (0, 1))
    xrb = jax.lax.broadcast_in_dim(xr, (bb, h, d2), (0, 1))
    lane = jax.lax.broadcasted_iota(jnp.int32, (bb, h, d2), 2)
    xe = jnp.where(lane < d, xlb, xrb)
    o_ref[...] = xe * w_ref[...][None, :, :]


def kernel(x, emb_weight):
    B, L = x.shape
    D = emb_weight.shape[1]
    H = L // 2
    wp = emb_weight[1:L + 1].reshape(H, 2 * D)
    xl = x[:, 0::2]
    xr = x[:, 1::2]
    out = pl.pallas_call(
        _embed_block,
        grid=(B // _BB,),
        in_specs=[
            pl.BlockSpec((_BB, H), lambda i: (i, 0)),
            pl.BlockSpec((_BB, H), lambda i: (i, 0)),
            pl.BlockSpec((H, 2 * D), lambda i: (0, 0)),
        ],
        out_specs=pl.BlockSpec((_BB, H, 2 * D), lambda i: (i, 0, 0)),
        out_shape=jax.ShapeDtypeStruct((B, H, 2 * D), x.dtype),
    )(xl, xr, wp)
    return out.reshape(B, L, D)


# speedup vs baseline: 22.9928x; 1.0012x over previous
import jax
import jax.numpy as jnp
from jax.experimental import pallas as pl

_BB = 512


def _embed_block(xl_ref, xr_ref, w_ref, o_ref):
    bb, h, d2 = o_ref.shape
    d = d2 // 2
    xl = xl_ref[...]
    xr = xr_ref[...]
    xl = jnp.where(jnp.isnan(xl), 0.0, xl)
    xr = jnp.where(jnp.isnan(xr), 0.0, xr)
    xlb = jax.lax.broadcast_in_dim(xl, (bb, h, d2), (0, 1))
    xrb = jax.lax.broadcast_in_dim(xr, (bb, h, d2), (0, 1))
    lane = jax.lax.broadcasted_iota(jnp.int32, (bb, h, d2), 2)
    xe = jnp.where(lane < d, xlb, xrb)
    o_ref[...] = xe * w_ref[...][None, :, :]


def kernel(x, emb_weight):
    B, L = x.shape
    D = emb_weight.shape[1]
    H = L // 2
    wp = emb_weight[1:L + 1].reshape(H, 2 * D)
    xl = x[:, 0::2]
    xr = x[:, 1::2]
    out = pl.pallas_call(
        _embed_block,
        grid=(B // _BB,),
        in_specs=[
            pl.BlockSpec((_BB, H), lambda i: (i, 0)),
            pl.BlockSpec((_BB, H), lambda i: (i, 0)),
            pl.BlockSpec((H, 2 * D), lambda i: (0, 0)),
        ],
        out_specs=pl.BlockSpec((_BB, H, 2 * D), lambda i: (i, 0, 0)),
        out_shape=jax.ShapeDtypeStruct((B, H, 2 * D), x.dtype),
    )(xl, xr, wp)
    return out.reshape(B, L, D)
